# Initial kernel scaffold; baseline (speedup 1.0000x reference)
import jax
import jax.numpy as jnp
from jax.experimental import pallas as pl


def kernel(x):
    raise NotImplementedError("write your pallas kernel here")


# speedup vs baseline: 21.9371x; 21.9371x over previous
import functools

import jax
import jax.numpy as jnp
from jax.experimental import pallas as pl
from jax.experimental.pallas import tpu as pltpu

_IN_SHAPE = (96, 512)
_SL_LO, _SL_HI = 128, 300
_EPS_LOG = 0.001
_INT_MIN = -(2**31)


def _select_normalize_kernel(k, x_ref, o_ref, key_ref):
    x = x_ref[...]
    xmin = jnp.min(x)
    xmax = jnp.max(x)

    y = jax.lax.bitcast_convert_type(x, jnp.int32)
    key_ref[...] = jnp.where(y >= 0, y, _INT_MIN - y)

    def body(i, prefix_ub):
        b = jnp.int32(31) - i
        trial_ub = prefix_ub | jnp.left_shift(jnp.int32(1), b)
        trial_s = trial_ub ^ _INT_MIN
        cnt = jnp.sum((key_ref[...] < trial_s).astype(jnp.int32))
        return jnp.where(cnt <= k, trial_ub, prefix_ub)

    res_ub = jax.lax.fori_loop(0, 32, body, jnp.int32(0))
    skey = res_ub ^ _INT_MIN
    ybits = jnp.where(skey >= 0, skey, _INT_MIN - skey)
    eps = jax.lax.bitcast_convert_type(ybits, jnp.float32)

    lo = jnp.maximum(eps, jnp.float32(_EPS_LOG))
    vmin = jnp.log10(jnp.maximum(xmin, lo))
    vmax = jnp.log10(jnp.maximum(xmax, lo))
    scale = jnp.float32(1.0) / (vmax - vmin)
    o_ref[...] = (jnp.log10(jnp.maximum(x, lo)) - vmin) * scale


@jax.jit
def kernel(x):
    b = x.size // (_IN_SHAPE[0] * _IN_SHAPE[1])
    xs = x.reshape((b,) + _IN_SHAPE)[:, :, _SL_LO:_SL_HI]
    out_shape = xs.shape
    n = xs.size
    rows = n // 128
    xs2 = xs.reshape(rows, 128)
    k = int(0.1 * n)

    out = pl.pallas_call(
        functools.partial(_select_normalize_kernel, k),
        out_shape=jax.ShapeDtypeStruct((rows, 128), jnp.float32),
        scratch_shapes=[pltpu.VMEM((rows, 128), jnp.int32)],
    )(xs2)
    return out.reshape(out_shape)


# speedup vs baseline: 25.8577x; 1.1787x over previous
import functools

import jax
import jax.numpy as jnp
from jax.experimental import pallas as pl
from jax.experimental.pallas import tpu as pltpu

_IN_SHAPE = (96, 512)
_SL_LO, _SL_HI = 128, 300
_EPS_LOG = 0.001
_INT_MIN = -(2**31)


def _select_normalize_kernel(k, x_ref, o_ref, key_ref):
    x = x_ref[...]
    xmin = jnp.min(x)
    xmax = jnp.max(x)

    y = jax.lax.bitcast_convert_type(x, jnp.int32)
    key_ref[...] = jnp.where(y >= 0, y, _INT_MIN - y)

    def body(i, prefix_ub):
        b = jnp.int32(31) - i
        trial_ub = prefix_ub | jnp.left_shift(jnp.int32(1), b)
        trial_s = trial_ub ^ _INT_MIN
        cnt = jnp.sum((key_ref[...] < trial_s).astype(jnp.int32))
        return jnp.where(cnt <= k, trial_ub, prefix_ub)

    res_ub = jax.lax.fori_loop(0, 22, body, jnp.int32(0))
    skey = res_ub ^ _INT_MIN
    ybits = jnp.where(skey >= 0, skey, _INT_MIN - skey)
    eps = jax.lax.bitcast_convert_type(ybits, jnp.float32)

    lo = jnp.maximum(eps, jnp.float32(_EPS_LOG))
    vmin = jnp.log10(jnp.maximum(xmin, lo))
    vmax = jnp.log10(jnp.maximum(xmax, lo))
    scale = jnp.float32(1.0) / (vmax - vmin)
    o_ref[...] = (jnp.log10(jnp.maximum(x, lo)) - vmin) * scale


@jax.jit
def kernel(x):
    b = x.size // (_IN_SHAPE[0] * _IN_SHAPE[1])
    xs = x.reshape((b,) + _IN_SHAPE)[:, :, _SL_LO:_SL_HI]
    out_shape = xs.shape
    n = xs.size
    rows = n // 128
    xs2 = xs.reshape(rows, 128)
    k = int(0.1 * n)

    out = pl.pallas_call(
        functools.partial(_select_normalize_kernel, k),
        out_shape=jax.ShapeDtypeStruct((rows, 128), jnp.float32),
        scratch_shapes=[pltpu.VMEM((rows, 128), jnp.int32)],
    )(xs2)
    return out.reshape(out_shape)
